# baseline (speedup 1.0000x reference)
import functools

import jax
import jax.numpy as jnp
import numpy as np
from jax import lax
from jax.experimental import pallas as pl
from jax.experimental.pallas import tpu as pltpu
from jax.experimental.pallas import tpu_sc as plsc

_NUM_CORES = 2
_NUM_SUBCORES = 16
_NW = _NUM_CORES * _NUM_SUBCORES
_GATHER = 128
_G = 2
_CHUNK = _G * _GATHER
_LANES = 16


def _make_kernel(B, V, D):
    nfull = B // _CHUNK
    tail = B - nfull * _CHUNK
    kmax = -(-nfull // _NW)
    t_rem_sz = (B - (B // _CHUNK) * _CHUNK) % _GATHER
    n_stage8 = V // 8
    v_rem = V - n_stage8 * 8
    scale = float(np.float32(1.0) / np.sqrt(np.float32(V)))

    mesh = plsc.VectorSubcoreMesh(core_axis_name="c", subcore_axis_name="s")

    @functools.partial(
        pl.kernel,
        mesh=mesh,
        out_type=jax.ShapeDtypeStruct((B, D), jnp.float32),
        scratch_types=[
            pltpu.VMEM((2, _G, _GATHER), jnp.int32),
            pltpu.VMEM((2, _CHUNK, D), jnp.float32),
            pltpu.VMEM((8, D), jnp.float32),
            pltpu.VMEM((max(t_rem_sz, 8),), jnp.int32),
            pltpu.VMEM_SHARED((V, D), jnp.float32),
            pltpu.SemaphoreType.DMA,
            pltpu.SemaphoreType.DMA,
            pltpu.SemaphoreType.DMA,
            pltpu.SemaphoreType.DMA,
            pltpu.SemaphoreType.DMA,
        ],
    )
    def k(emb_hbm, idx_hbm, out_hbm, idx_v, rows_v, tab_v, tidx_v, w_sp,
          gsem, isem0, isem1, ssem0, ssem1):
        isem = (isem0, isem1)
        ssem = (ssem0, ssem1)
        s = lax.axis_index("s")
        wid = s * _NUM_CORES + lax.axis_index("c")

        def stage(r0, nr):
            pltpu.sync_copy(emb_hbm.at[pl.ds(r0, nr)], tab_v.at[pl.ds(0, nr)])
            for dr in range(nr):
                for j in range(D // _LANES):
                    col = pl.ds(j * _LANES, _LANES)
                    tab_v[dr, col] = tab_v[dr, col] * scale
            pltpu.sync_copy(tab_v.at[pl.ds(0, nr)], w_sp.at[pl.ds(r0, nr)])

        @pl.when(s < n_stage8)
        def _():
            stage(s * 8, 8)

        if v_rem:
            @pl.when(s == n_stage8)
            def _():
                stage(n_stage8 * 8, v_rem)

        plsc.subcore_barrier()

        def cid(k_):
            return wid + k_ * _NW

        def idx_desc(k_, b, g):
            return pltpu.make_async_copy(
                idx_hbm.at[pl.ds(cid(k_) * _CHUNK + g * _GATHER, _GATHER)],
                idx_v.at[b, g], isem[b])

        def idx_start_b(k_, b):
            @pl.when(cid(k_) < nfull)
            def _():
                for g in range(_G):
                    idx_desc(k_, b, g).start()

        def gather_desc(b, g):
            return pltpu.make_async_copy(
                w_sp.at[idx_v.at[b, g]],
                rows_v.at[b, pl.ds(g * _GATHER, _GATHER)], gsem)

        def store_desc_b(k_, b):
            return pltpu.make_async_copy(
                rows_v.at[b], out_hbm.at[pl.ds(cid(k_) * _CHUNK, _CHUNK)],
                ssem[b])

        def process(k_, b, drain):
            valid = cid(k_) < nfull

            if drain:
                @pl.when(valid)
                def _():
                    store_desc_b(k_ - 2, b).wait()

            @pl.when(valid)
            def _():
                for g in range(_G):
                    idx_desc(k_, b, g).wait()
                for g in range(_G):
                    gather_desc(b, g).start()
                for g in range(_G):
                    gather_desc(b, g).wait()
                store_desc_b(k_, b).start()

            idx_start_b(k_ + 2, b)

        idx_start_b(0, 0)
        idx_start_b(1, 1)
        process(0, 0, drain=False)
        process(1, 1, drain=False)

        npairs = (kmax - 2) // 2

        @pl.loop(1, 1 + npairs)
        def _(t):
            process(2 * t, 0, drain=True)
            process(2 * t + 1, 1, drain=True)

        for k_ in range(2 + 2 * npairs, kmax):
            process(k_, k_ % 2, drain=True)

        for k_ in range(max(0, kmax - 3), kmax):
            @pl.when((cid(k_) < nfull) & (cid(k_ + 2) >= nfull))
            def _():
                store_desc_b(k_, k_ % 2).wait()

        if tail:
            t_full = tail // _GATHER
            t_rem = tail - t_full * _GATHER

            @pl.when(wid == (nfull % _NW))
            def _():
                base = nfull * _CHUNK
                for g in range(t_full):
                    pltpu.sync_copy(
                        idx_hbm.at[pl.ds(base + g * _GATHER, _GATHER)],
                        idx_v.at[0, g])
                if t_rem:
                    pltpu.sync_copy(
                        idx_hbm.at[pl.ds(base + t_full * _GATHER, t_rem)],
                        tidx_v)
                for g in range(t_full):
                    gather_desc(0, g).start()
                if t_rem:
                    pltpu.make_async_copy(
                        w_sp.at[tidx_v],
                        rows_v.at[0, pl.ds(t_full * _GATHER, t_rem)],
                        gsem).start()
                for g in range(t_full):
                    gather_desc(0, g).wait()
                if t_rem:
                    pltpu.make_async_copy(
                        w_sp.at[tidx_v],
                        rows_v.at[0, pl.ds(t_full * _GATHER, t_rem)],
                        gsem).wait()
                pltpu.sync_copy(rows_v.at[0, pl.ds(0, tail)],
                                out_hbm.at[pl.ds(base, tail)])

    return k


def kernel(node_species, embeddings):
    V, D = embeddings.shape
    B = node_species.shape[0]
    idx = node_species.astype(jnp.int32)
    return _make_kernel(B, V, D)(embeddings, idx)


# speedup vs baseline: 1.0295x; 1.0295x over previous
import functools

import jax
import jax.numpy as jnp
import numpy as np
from jax import lax
from jax.experimental import pallas as pl
from jax.experimental.pallas import tpu as pltpu
from jax.experimental.pallas import tpu_sc as plsc

_NUM_CORES = 2
_NUM_SUBCORES = 16
_NW = _NUM_CORES * _NUM_SUBCORES
_GATHER = 128
_G = 1
_CHUNK = _G * _GATHER
_LANES = 16


def _make_kernel(B, V, D):
    nfull = B // _CHUNK
    tail = B - nfull * _CHUNK
    kmax = -(-nfull // _NW)
    t_rem_sz = (B - (B // _CHUNK) * _CHUNK) % _GATHER
    n_stage8 = V // 8
    v_rem = V - n_stage8 * 8
    scale = float(np.float32(1.0) / np.sqrt(np.float32(V)))

    mesh = plsc.VectorSubcoreMesh(core_axis_name="c", subcore_axis_name="s")

    @functools.partial(
        pl.kernel,
        mesh=mesh,
        out_type=jax.ShapeDtypeStruct((B, D), jnp.float32),
        scratch_types=[
            pltpu.VMEM((2, _G, _GATHER), jnp.int32),
            pltpu.VMEM((2, _CHUNK, D), jnp.float32),
            pltpu.VMEM((8, D), jnp.float32),
            pltpu.VMEM((max(t_rem_sz, 8),), jnp.int32),
            pltpu.VMEM_SHARED((V, D), jnp.float32),
            pltpu.SemaphoreType.DMA,
            pltpu.SemaphoreType.DMA,
            pltpu.SemaphoreType.DMA,
            pltpu.SemaphoreType.DMA,
            pltpu.SemaphoreType.DMA,
        ],
    )
    def k(emb_hbm, idx_hbm, out_hbm, idx_v, rows_v, tab_v, tidx_v, w_sp,
          gsem, isem0, isem1, ssem0, ssem1):
        isem = (isem0, isem1)
        ssem = (ssem0, ssem1)
        s = lax.axis_index("s")
        wid = s * _NUM_CORES + lax.axis_index("c")

        def stage(r0, nr):
            pltpu.sync_copy(emb_hbm.at[pl.ds(r0, nr)], tab_v.at[pl.ds(0, nr)])
            for dr in range(nr):
                for j in range(D // _LANES):
                    col = pl.ds(j * _LANES, _LANES)
                    tab_v[dr, col] = tab_v[dr, col] * scale
            pltpu.sync_copy(tab_v.at[pl.ds(0, nr)], w_sp.at[pl.ds(r0, nr)])

        @pl.when(s < n_stage8)
        def _():
            stage(s * 8, 8)

        if v_rem:
            @pl.when(s == n_stage8)
            def _():
                stage(n_stage8 * 8, v_rem)

        plsc.subcore_barrier()

        def cid(k_):
            return wid + k_ * _NW

        def idx_desc(k_, b, g):
            return pltpu.make_async_copy(
                idx_hbm.at[pl.ds(cid(k_) * _CHUNK + g * _GATHER, _GATHER)],
                idx_v.at[b, g], isem[b])

        def idx_start_b(k_, b):
            @pl.when(cid(k_) < nfull)
            def _():
                for g in range(_G):
                    idx_desc(k_, b, g).start()

        def gather_desc(b, g):
            return pltpu.make_async_copy(
                w_sp.at[idx_v.at[b, g]],
                rows_v.at[b, pl.ds(g * _GATHER, _GATHER)], gsem)

        def store_desc_b(k_, b):
            return pltpu.make_async_copy(
                rows_v.at[b], out_hbm.at[pl.ds(cid(k_) * _CHUNK, _CHUNK)],
                ssem[b])

        def process(k_, b, drain):
            valid = cid(k_) < nfull

            if drain:
                @pl.when(valid)
                def _():
                    store_desc_b(k_ - 2, b).wait()

            @pl.when(valid)
            def _():
                for g in range(_G):
                    idx_desc(k_, b, g).wait()
                for g in range(_G):
                    gather_desc(b, g).start()
                for g in range(_G):
                    gather_desc(b, g).wait()
                store_desc_b(k_, b).start()

            idx_start_b(k_ + 2, b)

        idx_start_b(0, 0)
        idx_start_b(1, 1)
        process(0, 0, drain=False)
        process(1, 1, drain=False)

        npairs = (kmax - 2) // 2

        @pl.loop(1, 1 + npairs)
        def _(t):
            process(2 * t, 0, drain=True)
            process(2 * t + 1, 1, drain=True)

        for k_ in range(2 + 2 * npairs, kmax):
            process(k_, k_ % 2, drain=True)

        for k_ in range(max(0, kmax - 3), kmax):
            @pl.when((cid(k_) < nfull) & (cid(k_ + 2) >= nfull))
            def _():
                store_desc_b(k_, k_ % 2).wait()

        if tail:
            t_full = tail // _GATHER
            t_rem = tail - t_full * _GATHER

            @pl.when(wid == (nfull % _NW))
            def _():
                base = nfull * _CHUNK
                for g in range(t_full):
                    pltpu.sync_copy(
                        idx_hbm.at[pl.ds(base + g * _GATHER, _GATHER)],
                        idx_v.at[0, g])
                if t_rem:
                    pltpu.sync_copy(
                        idx_hbm.at[pl.ds(base + t_full * _GATHER, t_rem)],
                        tidx_v)
                for g in range(t_full):
                    gather_desc(0, g).start()
                if t_rem:
                    pltpu.make_async_copy(
                        w_sp.at[tidx_v],
                        rows_v.at[0, pl.ds(t_full * _GATHER, t_rem)],
                        gsem).start()
                for g in range(t_full):
                    gather_desc(0, g).wait()
                if t_rem:
                    pltpu.make_async_copy(
                        w_sp.at[tidx_v],
                        rows_v.at[0, pl.ds(t_full * _GATHER, t_rem)],
                        gsem).wait()
                pltpu.sync_copy(rows_v.at[0, pl.ds(0, tail)],
                                out_hbm.at[pl.ds(base, tail)])

    return k


def kernel(node_species, embeddings):
    V, D = embeddings.shape
    B = node_species.shape[0]
    idx = node_species.astype(jnp.int32)
    return _make_kernel(B, V, D)(embeddings, idx)
